# Initial kernel scaffold; baseline (speedup 1.0000x reference)
import jax
import jax.numpy as jnp
from jax.experimental import pallas as pl


def kernel(rpn_cls_score, gt_boxes, im_info, num_boxes):
    raise NotImplementedError("write your pallas kernel here")


# speedup vs baseline: 6.4307x; 6.4307x over previous
import functools

import numpy as np
import jax
import jax.numpy as jnp
from jax import lax
from jax.experimental import pallas as pl
from jax.experimental.pallas import tpu as pltpu
from jax.experimental.pallas import tpu_sc as plsc


def _anchor_base():
    ratios = np.array([0.5, 1.0, 2.0], dtype=np.float64)
    scales = np.array([8.0, 16.0, 32.0], dtype=np.float64)
    base = np.array([1.0, 1.0, 16.0, 16.0], dtype=np.float64) - 1.0
    w = base[2] - base[0] + 1.0
    h = base[3] - base[1] + 1.0
    x_ctr = base[0] + 0.5 * (w - 1.0)
    y_ctr = base[1] + 0.5 * (h - 1.0)
    size = w * h
    size_ratios = size / ratios
    ws_r = np.round(np.sqrt(size_ratios))
    hs_r = np.round(ws_r * ratios)
    rows = []
    for i in range(len(ratios)):
        for s in scales:
            wss = ws_r[i] * s
            hss = hs_r[i] * s
            rows.append([x_ctr - 0.5 * (wss - 1.0), y_ctr - 0.5 * (hss - 1.0),
                         x_ctr + 0.5 * (wss - 1.0), y_ctr + 0.5 * (hss - 1.0)])
    return np.array(rows, dtype=np.float32)


_AB = _anchor_base()

H = 50
W = 50
A = 9
K = 20
HW = H * W
FEAT_STRIDE = 16
NT = 16
CHUNK = 256
VPT = CHUNK // 16
NJ = A * VPT
HWP = NT * CHUNK

NROW = 9 + 3 * (A * 4)

_OUT_TYPE = jax.ShapeDtypeStruct((NROW, HWP), jnp.float32)

_SCRATCH_TYPES = [
    pltpu.VMEM((4, K * 16), jnp.float32),
    pltpu.VMEM((1, 32), jnp.float32),
    pltpu.VMEM((1, 16), jnp.int32),
    pltpu.VMEM((1, NJ * K * 16), jnp.float32),
    pltpu.VMEM((1, K * 16), jnp.float32),
    pltpu.VMEM((1, NJ * 16), jnp.float32),
    pltpu.VMEM((1, NJ * 16), jnp.int32),
    pltpu.VMEM((1, NJ * 16), jnp.int32),
    pltpu.VMEM((1, NJ * 16), jnp.float32),
    pltpu.VMEM((1, K * 16), jnp.float32),
    pltpu.VMEM((1, K * 16), jnp.float32),
    pltpu.VMEM((NROW, CHUNK), jnp.float32),
    pltpu.VMEM_SHARED((NT, 1, K * 16), jnp.float32),
]


def _gather_gt(gtb_v, am, c, iota):
    return plsc.load_gather(gtb_v, [jnp.full((16,), c, jnp.int32),
                                    am * 16 + iota])


def _ln(x):
    bits = plsc.bitcast(x, jnp.int32)
    e = (bits >> 23) - 127
    m = plsc.bitcast((bits & 0x007FFFFF) | 0x3F800000, jnp.float32)
    big = m > 1.4142135623730951
    m = jnp.where(big, m * 0.5, m)
    e = jnp.where(big, e + 1, e).astype(jnp.float32)
    s = (m - 1.0) / (m + 1.0)
    z = s * s
    ln_m = s * (2.0 + z * (0.6666666666666666 + z * (0.4 + z * 0.2857142857142857)))
    return e * 0.6931471805599453 + ln_m


def _sc_body(gtb, imwh, nbv, out_hbm,
             gtb_v, imwh_v, nbv_v, ov_s, colmax, maxov_s, argmax_s,
             inside_s, labels_s, gtmax_v, cm_tmp, stage_out, stage_cm):
    tid = lax.axis_index("s")
    start = tid * CHUNK
    pltpu.sync_copy(gtb, gtb_v)
    pltpu.sync_copy(imwh, imwh_v)
    pltpu.sync_copy(nbv, nbv_v)
    imh = imwh_v[0, pl.ds(0, 16)]
    imw = imwh_v[0, pl.ds(16, 16)]
    nb = nbv_v[0, :]
    iota = lax.iota(jnp.int32, 16)

    def cm_init(k, _):
        colmax[0, pl.ds(k * 16, 16)] = jnp.full((16,), -2.0, jnp.float32)
        return 0

    lax.fori_loop(0, K, cm_init, 0)

    for a in range(A):
        bx1, by1, bx2, by2 = (float(_AB[a, 0]), float(_AB[a, 1]),
                              float(_AB[a, 2]), float(_AB[a, 3]))

        def vbody(v, _, bx1=bx1, by1=by1, bx2=bx2, by2=by2, a=a):
            j = a * VPT + v
            hw = start + v * 16 + iota
            wi = hw % W
            hi = hw // W
            sx = (wi * FEAT_STRIDE).astype(jnp.float32)
            sy = (hi * FEAT_STRIDE).astype(jnp.float32)
            ax1 = sx + bx1
            ay1 = sy + by1
            ax2 = sx + bx2
            ay2 = sy + by2
            ins = (ax1 >= 0.0) & (ay1 >= 0.0) & (ax2 < imw) & (ay2 < imh) & (hw < HW)
            aw = ax2 - ax1 + 1.0
            ah = ay2 - ay1 + 1.0
            area = aw * ah
            inside_s[0, pl.ds(j * 16, 16)] = jnp.where(ins, 1, 0)
            joff = j * (K * 16)

            def kbody(k, carry):
                mx, am = carry
                gx1 = gtb_v[0, pl.ds(k * 16, 16)]
                gy1 = gtb_v[1, pl.ds(k * 16, 16)]
                gx2 = gtb_v[2, pl.ds(k * 16, 16)]
                gy2 = gtb_v[3, pl.ds(k * 16, 16)]
                ix1 = jnp.maximum(ax1, gx1)
                iy1 = jnp.maximum(ay1, gy1)
                ix2 = jnp.minimum(ax2, gx2)
                iy2 = jnp.minimum(ay2, gy2)
                iw = jnp.maximum(ix2 - ix1 + 1.0, 0.0)
                ih = jnp.maximum(iy2 - iy1 + 1.0, 0.0)
                inter = iw * ih
                ag = (gx2 - gx1 + 1.0) * (gy2 - gy1 + 1.0)
                ua = area + ag - inter
                ov = inter / ua
                ov = jnp.where(ins & (k < nb), ov, -1.0)
                ov_s[0, pl.ds(joff + k * 16, 16)] = ov
                colmax[0, pl.ds(k * 16, 16)] = jnp.maximum(
                    colmax[0, pl.ds(k * 16, 16)], ov)
                am = jnp.where(ov > mx, k, am)
                mx = jnp.maximum(mx, ov)
                return mx, am

            mx, am = lax.fori_loop(
                0, K, kbody,
                (jnp.full((16,), -2.0, jnp.float32), jnp.zeros((16,), jnp.int32)))
            maxov_s[0, pl.ds(j * 16, 16)] = mx
            argmax_s[0, pl.ds(j * 16, 16)] = am
            return 0

        lax.fori_loop(0, VPT, vbody, 0)

    pltpu.sync_copy(colmax, stage_cm.at[tid])
    plsc.subcore_barrier()

    def gk_init(k, _):
        gtmax_v[0, pl.ds(k * 16, 16)] = jnp.full((16,), -2.0, jnp.float32)
        return 0

    lax.fori_loop(0, K, gk_init, 0)

    def tmax(t, _):
        pltpu.sync_copy(stage_cm.at[t], cm_tmp)

        def kmax(k, _):
            gtmax_v[0, pl.ds(k * 16, 16)] = jnp.maximum(
                gtmax_v[0, pl.ds(k * 16, 16)], cm_tmp[0, pl.ds(k * 16, 16)])
            return 0

        lax.fori_loop(0, K, kmax, 0)
        return 0

    lax.fori_loop(0, NT, tmax, 0)

    def gk_bcast(k, _):
        g = jnp.max(gtmax_v[0, pl.ds(k * 16, 16)])
        gtmax_v[0, pl.ds(k * 16, 16)] = jnp.full((16,), g)
        return 0

    lax.fori_loop(0, K, gk_bcast, 0)

    def p2(j, cnt):
        mx = maxov_s[0, pl.ds(j * 16, 16)]
        ins = inside_s[0, pl.ds(j * 16, 16)] == 1
        joff = j * (K * 16)

        def kb(k, kp):
            ov = ov_s[0, pl.ds(joff + k * 16, 16)]
            gv = gtmax_v[0, pl.ds(k * 16, 16)]
            return jnp.where((ov == gv) & (gv > 0.0), 1, kp)

        kp = lax.fori_loop(0, K, kb, jnp.zeros((16,), jnp.int32))
        lab = jnp.where(ins & (mx < 0.3), 0.0, -1.0)
        lab = jnp.where(kp == 1, 1.0, lab)
        lab = jnp.where(ins & (mx >= 0.7), 1.0, lab)
        labels_s[0, pl.ds(j * 16, 16)] = lab
        return cnt + jnp.where(lab >= 0.0, 1.0, 0.0)

    cnt = lax.fori_loop(0, NJ, p2, jnp.zeros((16,), jnp.float32))
    plsc.subcore_barrier()
    colmax[0, pl.ds(0, 16)] = cnt
    pltpu.sync_copy(colmax, stage_cm.at[tid])
    plsc.subcore_barrier()

    def tsum(t, acc):
        pltpu.sync_copy(stage_cm.at[t], cm_tmp)
        return acc + cm_tmp[0, pl.ds(0, 16)]

    tot = lax.fori_loop(0, NT, tsum, jnp.zeros((16,), jnp.float32))
    nev = jnp.full((16,), jnp.sum(tot))
    invv = 1.0 / jnp.maximum(nev, 1.0)

    for a in range(A):
        bx1, by1, bx2, by2 = (float(_AB[a, 0]), float(_AB[a, 1]),
                              float(_AB[a, 2]), float(_AB[a, 3]))

        def v3(v, _, bx1=bx1, by1=by1, bx2=bx2, by2=by2, a=a):
            j = a * VPT + v
            hw = start + v * 16 + iota
            wi = hw % W
            hi = hw // W
            sx = (wi * FEAT_STRIDE).astype(jnp.float32)
            sy = (hi * FEAT_STRIDE).astype(jnp.float32)
            ax1 = sx + bx1
            ay1 = sy + by1
            ax2 = sx + bx2
            ay2 = sy + by2
            aw = ax2 - ax1 + 1.0
            ah = ay2 - ay1 + 1.0
            ectx = ax1 + 0.5 * (aw - 1.0)
            ecty = ay1 + 0.5 * (ah - 1.0)
            am = argmax_s[0, pl.ds(j * 16, 16)]
            gx1 = _gather_gt(gtb_v, am, 0, iota)
            gy1 = _gather_gt(gtb_v, am, 1, iota)
            gx2 = _gather_gt(gtb_v, am, 2, iota)
            gy2 = _gather_gt(gtb_v, am, 3, iota)
            gtw = gx2 - gx1 + 1.0
            gth = gy2 - gy1 + 1.0
            gcx = gx1 + 0.5 * (gtw - 1.0)
            gcy = gy1 + 0.5 * (gth - 1.0)
            dx = (gcx - ectx) / aw
            dy = (gcy - ecty) / ah
            dw = _ln(gtw / aw)
            dh = _ln(gth / ah)
            lab = labels_s[0, pl.ds(j * 16, 16)]
            col = pl.ds(v * 16, 16)
            stage_out[a, col] = lab
            iwv = jnp.where(lab == 1.0, 1.0, 0.0)
            owv = jnp.where(lab >= 0.0, invv, jnp.zeros((16,), jnp.float32))
            vals = (dx, dy, dw, dh)
            for c in range(4):
                stage_out[A + a * 4 + c, col] = vals[c]
                stage_out[A + A * 4 + a * 4 + c, col] = iwv
                stage_out[A + 2 * A * 4 + a * 4 + c, col] = owv
            return 0

        lax.fori_loop(0, VPT, v3, 0)

    pltpu.sync_copy(stage_out, out_hbm.at[:, pl.ds(start, CHUNK)])


@functools.cache
def _build_sc_call():
    mesh = plsc.VectorSubcoreMesh(core_axis_name="c", subcore_axis_name="s",
                                  num_cores=1, num_subcores=NT)
    return pl.kernel(_sc_body, out_type=_OUT_TYPE, mesh=mesh,
                     scratch_types=_SCRATCH_TYPES,
                     compiler_params=pltpu.CompilerParams(
                         needs_layout_passes=False))


def kernel(rpn_cls_score, gt_boxes, im_info, num_boxes):
    gt = gt_boxes[0, :, :4].astype(jnp.float32)
    gtb = jnp.broadcast_to(gt.T[:, :, None], (4, K, 16)).reshape(4, K * 16)
    imwh = jnp.broadcast_to(im_info[0, :2].astype(jnp.float32)[:, None],
                            (2, 16)).reshape(1, 32)
    nbv = jnp.broadcast_to(num_boxes.astype(jnp.int32), (16,)).reshape(1, 16)
    out = _build_sc_call()(gtb, imwh, nbv)
    lbl = out[:A, :HW].reshape(1, 1, A * H, W)
    bt = out[A:A + 36, :HW].reshape(1, A * 4, H, W)
    biw = out[A + 36:A + 72, :HW].reshape(1, A * 4, H, W)
    bow = out[A + 72:A + 108, :HW].reshape(1, A * 4, H, W)
    return lbl, bt, biw, bow
